# baseline (speedup 1.0000x reference)
import functools

import jax
import jax.numpy as jnp
from jax import lax
from jax.experimental import pallas as pl
from jax.experimental.pallas import tpu as pltpu
from jax.experimental.pallas import tpu_sc as plsc

N_FIELDS = 26
VOCAB = 100000
EMBED_DIM = 32
BATCH = 16384

NUM_WORKERS = 32
B_PER_W = BATCH // NUM_WORKERS
CHUNK = 128
N_CHUNKS = B_PER_W // CHUNK
LANES = 16
NBUF = 4


def _body(inputs_hbm, tables_hbm, out_hbm, idx_all, rows, sems_g, sems_w):
    wid = lax.axis_index("s") * 2 + lax.axis_index("c")
    base = wid * B_PER_W
    row0 = wid * N_CHUNKS

    pltpu.sync_copy(inputs_hbm.at[:, pl.ds(row0, N_CHUNKS)], idx_all)

    for i in range(N_FIELDS):
        off = jnp.full((LANES,), i * VOCAB, dtype=jnp.int32)
        for r in range(N_CHUNKS):
            for c in range(CHUNK // LANES):
                sl = pl.ds(c * LANES, LANES)
                idx_all[i, r, sl] = idx_all[i, r, sl] + off

    def fire_gather(i):
        b = i % NBUF
        return [
            pltpu.async_copy(
                tables_hbm.at[idx_all.at[i, c]],
                rows.at[b, pl.ds(c * CHUNK, CHUNK)],
                sems_g[b],
            )
            for c in range(N_CHUNKS)
        ]

    def fire_write(i):
        b = i % NBUF
        return pltpu.async_copy(
            rows.at[b],
            out_hbm.at[i, pl.ds(base, B_PER_W)],
            sems_w[b],
        )

    gathers = {i: fire_gather(i) for i in range(NBUF - 1)}
    writes = {}
    for j in range(N_FIELDS):
        nxt = j + NBUF - 1
        if nxt < N_FIELDS:
            if j >= 1:
                writes.pop(j - 1).wait()
            gathers[nxt] = fire_gather(nxt)
        for h in gathers.pop(j):
            h.wait()
        writes[j] = fire_write(j)
    for j in sorted(writes):
        writes.pop(j).wait()


@jax.jit
def _lookup(inputs3, tables_flat):
    mesh = plsc.VectorSubcoreMesh(core_axis_name="c", subcore_axis_name="s")
    f = functools.partial(
        pl.kernel,
        mesh=mesh,
        out_type=jax.ShapeDtypeStruct((N_FIELDS, BATCH, EMBED_DIM), jnp.float32),
        scratch_types=[
            pltpu.VMEM((N_FIELDS, N_CHUNKS, CHUNK), jnp.int32),
            pltpu.VMEM((NBUF, B_PER_W, EMBED_DIM), jnp.float32),
            [pltpu.SemaphoreType.DMA] * NBUF,
            [pltpu.SemaphoreType.DMA] * NBUF,
        ],
        compiler_params=pltpu.CompilerParams(use_tc_tiling_on_sc=False),
    )(_body)
    return f(inputs3, tables_flat)


def kernel(inputs, tables):
    inputs3 = inputs.astype(jnp.int32).reshape(N_FIELDS, BATCH // CHUNK, CHUNK)
    tables_flat = tables.reshape(N_FIELDS * VOCAB, EMBED_DIM)
    out3 = _lookup(inputs3, tables_flat)
    return out3.transpose(1, 0, 2).reshape(BATCH, N_FIELDS * EMBED_DIM)


# speedup vs baseline: 1.0712x; 1.0712x over previous
import functools

import jax
import jax.numpy as jnp
from jax import lax
from jax.experimental import pallas as pl
from jax.experimental.pallas import tpu as pltpu
from jax.experimental.pallas import tpu_sc as plsc

N_FIELDS = 26
VOCAB = 100000
EMBED_DIM = 32
BATCH = 16384

NUM_WORKERS = 32
B_PER_W = BATCH // NUM_WORKERS
CHUNK = 128
N_CHUNKS = B_PER_W // CHUNK
LANES = 16
NBUF = 6


def _body(inputs_hbm, tables_hbm, out_hbm, idx_all, rows, sems_g, sems_w):
    wid = lax.axis_index("s") * 2 + lax.axis_index("c")
    base = wid * B_PER_W
    row0 = wid * N_CHUNKS

    pltpu.sync_copy(inputs_hbm.at[:, pl.ds(row0, N_CHUNKS)], idx_all)

    for i in range(N_FIELDS):
        off = jnp.full((LANES,), i * VOCAB, dtype=jnp.int32)
        for r in range(N_CHUNKS):
            for c in range(CHUNK // LANES):
                sl = pl.ds(c * LANES, LANES)
                idx_all[i, r, sl] = idx_all[i, r, sl] + off

    def fire_gather(i):
        b = i % NBUF
        return [
            pltpu.async_copy(
                tables_hbm.at[idx_all.at[i, c]],
                rows.at[b, pl.ds(c * CHUNK, CHUNK)],
                sems_g[b],
            )
            for c in range(N_CHUNKS)
        ]

    def fire_write(i):
        b = i % NBUF
        return pltpu.async_copy(
            rows.at[b],
            out_hbm.at[pl.ds(base, B_PER_W), pl.ds(i * EMBED_DIM, EMBED_DIM)],
            sems_w[b],
        )

    gathers = {i: fire_gather(i) for i in range(NBUF - 1)}
    writes = {}
    for j in range(N_FIELDS):
        nxt = j + NBUF - 1
        if nxt < N_FIELDS:
            if j >= 1:
                writes.pop(j - 1).wait()
            gathers[nxt] = fire_gather(nxt)
        for h in gathers.pop(j):
            h.wait()
        writes[j] = fire_write(j)
    for j in sorted(writes):
        writes.pop(j).wait()


@jax.jit
def _lookup(inputs3, tables_flat):
    mesh = plsc.VectorSubcoreMesh(core_axis_name="c", subcore_axis_name="s")
    f = functools.partial(
        pl.kernel,
        mesh=mesh,
        out_type=jax.ShapeDtypeStruct((BATCH, N_FIELDS * EMBED_DIM), jnp.float32),
        scratch_types=[
            pltpu.VMEM((N_FIELDS, N_CHUNKS, CHUNK), jnp.int32),
            pltpu.VMEM((NBUF, B_PER_W, EMBED_DIM), jnp.float32),
            [pltpu.SemaphoreType.DMA] * NBUF,
            [pltpu.SemaphoreType.DMA] * NBUF,
        ],
        compiler_params=pltpu.CompilerParams(use_tc_tiling_on_sc=False),
    )(_body)
    return f(inputs3, tables_flat)


def kernel(inputs, tables):
    inputs3 = inputs.astype(jnp.int32).reshape(N_FIELDS, BATCH // CHUNK, CHUNK)
    tables_flat = tables.reshape(N_FIELDS * VOCAB, EMBED_DIM)
    return _lookup(inputs3, tables_flat)


# speedup vs baseline: 1.0725x; 1.0012x over previous
import functools

import jax
import jax.numpy as jnp
from jax import lax
from jax.experimental import pallas as pl
from jax.experimental.pallas import tpu as pltpu
from jax.experimental.pallas import tpu_sc as plsc

N_FIELDS = 26
VOCAB = 100000
EMBED_DIM = 32
BATCH = 16384

NUM_WORKERS = 32
B_PER_W = BATCH // NUM_WORKERS
CHUNK = 128
N_CHUNKS = B_PER_W // CHUNK
LANES = 16
NBUF = 6


def _body(inputs_hbm, tables_hbm, out_hbm, idx_all, rows, sems_g, sems_w):
    wid = lax.axis_index("s") * 2 + lax.axis_index("c")
    base = wid * B_PER_W

    pltpu.sync_copy(inputs_hbm.at[:, wid], idx_all)

    for i in range(N_FIELDS):
        off = jnp.full((LANES,), i * VOCAB, dtype=jnp.int32)
        for c in range(B_PER_W // LANES):
            sl = pl.ds(c * LANES, LANES)
            idx_all[i, sl] = idx_all[i, sl] + off

    def fire_gather(i):
        b = i % NBUF
        return [
            pltpu.async_copy(
                tables_hbm.at[idx_all.at[i]],
                rows.at[b],
                sems_g[b],
            )
        ]

    def fire_write(i):
        b = i % NBUF
        return pltpu.async_copy(
            rows.at[b],
            out_hbm.at[pl.ds(base, B_PER_W), pl.ds(i * EMBED_DIM, EMBED_DIM)],
            sems_w[b],
        )

    gathers = {i: fire_gather(i) for i in range(NBUF - 1)}
    writes = {}
    for j in range(N_FIELDS):
        nxt = j + NBUF - 1
        if nxt < N_FIELDS:
            if j >= 1:
                writes.pop(j - 1).wait()
            gathers[nxt] = fire_gather(nxt)
        for h in gathers.pop(j):
            h.wait()
        writes[j] = fire_write(j)
    for j in sorted(writes):
        writes.pop(j).wait()


@jax.jit
def _lookup(inputs3, tables_flat):
    mesh = plsc.VectorSubcoreMesh(core_axis_name="c", subcore_axis_name="s")
    f = functools.partial(
        pl.kernel,
        mesh=mesh,
        out_type=jax.ShapeDtypeStruct((BATCH, N_FIELDS * EMBED_DIM), jnp.float32),
        scratch_types=[
            pltpu.VMEM((N_FIELDS, B_PER_W), jnp.int32),
            pltpu.VMEM((NBUF, B_PER_W, EMBED_DIM), jnp.float32),
            [pltpu.SemaphoreType.DMA] * NBUF,
            [pltpu.SemaphoreType.DMA] * NBUF,
        ],
        compiler_params=pltpu.CompilerParams(use_tc_tiling_on_sc=False),
    )(_body)
    return f(inputs3, tables_flat)


def kernel(inputs, tables):
    inputs3 = inputs.astype(jnp.int32).reshape(N_FIELDS, NUM_WORKERS, B_PER_W)
    tables_flat = tables.reshape(N_FIELDS * VOCAB, EMBED_DIM)
    return _lookup(inputs3, tables_flat)
